# baseline (speedup 1.0000x reference)
import functools
import jax
import jax.numpy as jnp
from jax import lax
from jax.experimental import pallas as pl
from jax.experimental.pallas import tpu as pltpu
from jax.experimental.pallas import tpu_sc as plsc

_W = 4


def _make_sc(Ro, D):
    info = plsc.get_sparse_core_info()
    NC, NS = info.num_cores, info.num_subcores
    NW = NC * NS
    rows_per_w = Ro // NW
    mesh = plsc.VectorSubcoreMesh(core_axis_name="c", subcore_axis_name="s")

    @functools.partial(
        pl.kernel,
        mesh=mesh,
        out_type=jax.ShapeDtypeStruct((Ro, D), jnp.float32),
        scratch_types=[pltpu.SemaphoreType.DMA],
    )
    def k(x_hbm, out_hbm, sem):
        wid = lax.axis_index("s") * NC + lax.axis_index("c")
        base = wid * rows_per_w
        pltpu.async_copy(
            x_hbm.at[pl.ds(base, rows_per_w), pl.ds(0, D)],
            out_hbm.at[pl.ds(base, rows_per_w)],
            sem,
        ).wait()

    return k


def kernel(x):
    B, S, D = x.shape
    So = S // _W
    x2 = x.reshape(B * So, _W * D)
    out = _make_sc(B * So, D)(x2)
    return out.reshape(B, So, D)


# speedup vs baseline: 6.3132x; 6.3132x over previous
import functools
import jax
import jax.numpy as jnp
from jax import lax
from jax.experimental import pallas as pl
from jax.experimental.pallas import tpu as pltpu
from jax.experimental.pallas import tpu_sc as plsc

_W = 4
_R = 16
_NBUF = 3


def _make_sc(Ro, D):
    info = plsc.get_sparse_core_info()
    NC, NS = info.num_cores, info.num_subcores
    NW = NC * NS
    rows_per_w = Ro // NW
    nch = rows_per_w // _R
    mesh = plsc.VectorSubcoreMesh(core_axis_name="c", subcore_axis_name="s")

    @functools.partial(
        pl.kernel,
        mesh=mesh,
        out_type=jax.ShapeDtypeStruct((Ro, D), jnp.float32),
        scratch_types=(
            [pltpu.VMEM((_R, D), jnp.float32) for _ in range(_NBUF)]
            + [pltpu.SemaphoreType.DMA for _ in range(2 * _NBUF)]
        ),
    )
    def k(x_hbm, out_hbm, *refs):
        bufs = refs[:_NBUF]
        sin = refs[_NBUF:2 * _NBUF]
        sout = refs[2 * _NBUF:]
        wid = lax.axis_index("s") * NC + lax.axis_index("c")
        base = wid * rows_per_w

        def src(c):
            return x_hbm.at[pl.ds(base + c * _R, _R), pl.ds(0, D)]

        def dst(c):
            return out_hbm.at[pl.ds(base + c * _R, _R)]

        for c in range(min(_NBUF, nch)):
            pltpu.async_copy(src(c), bufs[c % _NBUF], sin[c % _NBUF])
        for c in range(nch):
            b = c % _NBUF
            pltpu.make_async_copy(src(c), bufs[b], sin[b]).wait()
            pltpu.async_copy(bufs[b], dst(c), sout[b])
            nxt = c + _NBUF
            if nxt < nch:
                pltpu.make_async_copy(bufs[b], dst(c), sout[b]).wait()
                pltpu.async_copy(src(nxt), bufs[b], sin[b])
        for c in range(max(0, nch - _NBUF), nch):
            b = c % _NBUF
            pltpu.make_async_copy(bufs[b], dst(c), sout[b]).wait()

    return k


def kernel(x):
    B, S, D = x.shape
    So = S // _W
    x2 = x.reshape(B * So, _W * D)
    out = _make_sc(B * So, D)(x2)
    return out.reshape(B, So, D)


# speedup vs baseline: 27.4950x; 4.3552x over previous
import functools
import jax
import jax.numpy as jnp
from jax import lax
from jax.experimental import pallas as pl
from jax.experimental.pallas import tpu as pltpu
from jax.experimental.pallas import tpu_sc as plsc

_W = 4
_R = 16
_NBUF = 3


def _make_sc(Ro, D):
    info = plsc.get_sparse_core_info()
    NC, NS = info.num_cores, info.num_subcores
    NW = NC * NS
    rows_per_w = Ro // NW
    nch = rows_per_w // _R
    mesh = plsc.VectorSubcoreMesh(core_axis_name="c", subcore_axis_name="s")

    @functools.partial(
        pl.kernel,
        mesh=mesh,
        out_type=jax.ShapeDtypeStruct((Ro, D), jnp.float32),
        scratch_types=(
            [pltpu.VMEM((_R, D), jnp.float32) for _ in range(_NBUF)]
            + [pltpu.VMEM((rows_per_w,), jnp.int32)]
            + [pltpu.SemaphoreType.DMA for _ in range(2 * _NBUF)]
        ),
    )
    def k(x_hbm, idx_hbm, out_hbm, *refs):
        bufs = refs[:_NBUF]
        idx_v = refs[_NBUF]
        sin = refs[_NBUF + 1:_NBUF + 1 + _NBUF]
        sout = refs[_NBUF + 1 + _NBUF:]
        wid = lax.axis_index("s") * NC + lax.axis_index("c")
        base = wid * rows_per_w
        pltpu.sync_copy(idx_hbm.at[pl.ds(base, rows_per_w)], idx_v)

        def start_gather(c, b):
            pltpu.async_copy(
                x_hbm.at[idx_v.at[pl.ds(c * _R, _R)]], bufs[b], sin[b]
            )

        def gather_wait(c, b):
            pltpu.make_async_copy(
                x_hbm.at[idx_v.at[pl.ds(c * _R, _R)]], bufs[b], sin[b]
            ).wait()

        def dst(c):
            return out_hbm.at[pl.ds(base + c * _R, _R)]

        for c in range(min(_NBUF, nch)):
            start_gather(c, c % _NBUF)
        for c in range(nch):
            b = c % _NBUF
            gather_wait(c, b)
            pltpu.async_copy(bufs[b], dst(c), sout[b])
            nxt = c + _NBUF
            if nxt < nch:
                pltpu.make_async_copy(bufs[b], dst(c), sout[b]).wait()
                start_gather(nxt, b)
        for c in range(max(0, nch - _NBUF), nch):
            b = c % _NBUF
            pltpu.make_async_copy(bufs[b], dst(c), sout[b]).wait()

    return k


def kernel(x):
    B, S, D = x.shape
    So = S // _W
    x2 = x.reshape(B * S, D)
    idx = jnp.arange(0, B * S, _W, dtype=jnp.int32)
    out = _make_sc(B * So, D)(x2, idx)
    return out.reshape(B, So, D)


# speedup vs baseline: 27.7822x; 1.0104x over previous
import functools
import numpy as np
import jax
import jax.numpy as jnp
from jax import lax
from jax.experimental import pallas as pl
from jax.experimental.pallas import tpu as pltpu
from jax.experimental.pallas import tpu_sc as plsc

_W = 4
_R = 8
_NBUF = 6


def _make_sc(Ro, D):
    info = plsc.get_sparse_core_info()
    NC, NS = info.num_cores, info.num_subcores
    NW = NC * NS
    rows_per_w = Ro // NW
    nch = rows_per_w // _R
    mesh = plsc.VectorSubcoreMesh(core_axis_name="c", subcore_axis_name="s")

    @functools.partial(
        pl.kernel,
        mesh=mesh,
        out_type=jax.ShapeDtypeStruct((Ro, D), jnp.float32),
        scratch_types=(
            [pltpu.VMEM((_R, D), jnp.float32) for _ in range(_NBUF)]
            + [pltpu.VMEM((rows_per_w,), jnp.int32)]
            + [pltpu.SemaphoreType.DMA for _ in range(2 * _NBUF)]
        ),
    )
    def k(x_hbm, idx_hbm, out_hbm, *refs):
        bufs = refs[:_NBUF]
        idx_v = refs[_NBUF]
        sin = refs[_NBUF + 1:_NBUF + 1 + _NBUF]
        sout = refs[_NBUF + 1 + _NBUF:]
        wid = lax.axis_index("s") * NC + lax.axis_index("c")
        base = wid * rows_per_w
        pltpu.sync_copy(idx_hbm.at[pl.ds(base, rows_per_w)], idx_v)

        def start_gather(c, b):
            pltpu.async_copy(
                x_hbm.at[idx_v.at[pl.ds(c * _R, _R)]], bufs[b], sin[b]
            )

        def gather_wait(c, b):
            pltpu.make_async_copy(
                x_hbm.at[idx_v.at[pl.ds(c * _R, _R)]], bufs[b], sin[b]
            ).wait()

        def dst(c):
            return out_hbm.at[pl.ds(base + c * _R, _R)]

        for c in range(min(_NBUF, nch)):
            start_gather(c, c % _NBUF)
        for c in range(nch):
            b = c % _NBUF
            gather_wait(c, b)
            pltpu.async_copy(bufs[b], dst(c), sout[b])
            nxt = c + _NBUF
            if nxt < nch:
                pltpu.make_async_copy(bufs[b], dst(c), sout[b]).wait()
                start_gather(nxt, b)
        for c in range(max(0, nch - _NBUF), nch):
            b = c % _NBUF
            pltpu.make_async_copy(bufs[b], dst(c), sout[b]).wait()

    return k


def kernel(x):
    B, S, D = x.shape
    So = S // _W
    x2 = x.reshape(B * S, D)
    idx = np.arange(0, B * S, _W, dtype=np.int32)
    out = _make_sc(B * So, D)(x2, idx)
    return out.reshape(B, So, D)
